# baseline (speedup 1.0000x reference)
import functools

import jax
import jax.numpy as jnp
from jax import lax
from jax.experimental import pallas as pl
from jax.experimental.pallas import tpu as pltpu
from jax.experimental.pallas import tpu_sc as plsc

_P = 8
_BS = 512


def _sc_broadcast_add(inputs, table, s_rows):
    batch, seq, dim = inputs.shape
    info = plsc.get_sparse_core_info()
    nc, ns, nl = info.num_cores, info.num_subcores, info.num_lanes
    nw = nc * ns
    s_per_w = s_rows // nw
    n_chunks = s_per_w // _P
    n_iter = n_chunks * batch
    mesh = plsc.VectorSubcoreMesh(core_axis_name="c", subcore_axis_name="s")

    @functools.partial(
        pl.kernel,
        mesh=mesh,
        out_type=jax.ShapeDtypeStruct((batch, seq, dim), jnp.float32),
        scratch_types=(
            [pltpu.VMEM((_P, dim), jnp.float32) for _ in range(5)]
            + [pltpu.VMEM((_P, dim), jnp.float32) for _ in range(2)]
            + [pltpu.SemaphoreType.DMA for _ in range(12)]
        ),
    )
    def k(in_hbm, tab_hbm, out_hbm, *scr):
        bufs = scr[0:5]
        tabs = scr[5:7]
        lsem = scr[7:12]
        ssem = scr[12:17]
        tsem = scr[17:19]
        wid = lax.axis_index("s") * nc + lax.axis_index("c")
        s0 = wid * s_per_w

        def in_slice(i):
            c, b = divmod(i, batch)
            return in_hbm.at[b, pl.ds(s0 + c * _P, _P)]

        def out_slice(i):
            c, b = divmod(i, batch)
            return out_hbm.at[b, pl.ds(s0 + c * _P, _P)]

        def tab_slice(c):
            return tab_hbm.at[pl.ds(s0 + c * _P, _P)]

        def load(i):
            return pltpu.make_async_copy(in_slice(i), bufs[i % 5], lsem[i % 5])

        def store(i):
            return pltpu.make_async_copy(bufs[i % 5], out_slice(i), ssem[i % 5])

        def tabcp(c):
            return pltpu.make_async_copy(tab_slice(c), tabs[c % 2], tsem[c % 2])

        tabcp(0).start()
        if n_chunks > 1:
            tabcp(1).start()
        for i in range(min(4, n_iter)):
            load(i).start()
        for i in range(n_iter):
            c, b = divmod(i, batch)
            load(i).wait()
            if b == 0:
                tabcp(c).wait()
            buf = bufs[i % 5]
            tv = tabs[c % 2]

            def body(i2, carry, buf=buf, tv=tv):
                r = i2 // 2
                col = (i2 % 2) * (dim // 2)
                for u in range(dim // (2 * nl)):
                    sl = pl.ds(col + u * nl, nl)
                    buf[r, sl] = buf[r, sl] + tv[r, sl]
                return carry

            lax.fori_loop(0, 2 * _P, body, 0)
            store(i).start()
            if b == batch - 1 and c + 2 < n_chunks:
                tabcp(c + 2).start()
            if i + 4 < n_iter:
                if i >= 1:
                    store(i - 1).wait()
                load(i + 4).start()
        for i in range(max(0, n_iter - 5), n_iter):
            store(i).wait()

    return k(inputs, table)


def _tc_fill_rest(inputs, table, partial, row0):
    batch, seq, dim = inputs.shape
    rows = seq - row0
    blk0 = row0 // _BS

    def body(in_ref, tab_ref, alias_ref, out_ref):
        del alias_ref
        out_ref[...] = in_ref[...] + tab_ref[...]

    return pl.pallas_call(
        body,
        grid=(rows // _BS, batch),
        in_specs=[
            pl.BlockSpec((1, _BS, dim), lambda i, b: (b, i + blk0, 0)),
            pl.BlockSpec((_BS, dim), lambda i, b: (i + blk0, 0)),
            pl.BlockSpec(memory_space=pl.ANY),
        ],
        out_specs=pl.BlockSpec((1, _BS, dim), lambda i, b: (b, i + blk0, 0)),
        out_shape=jax.ShapeDtypeStruct((batch, seq, dim), jnp.float32),
        input_output_aliases={2: 0},
    )(inputs, table, partial)


def kernel(inputs, position_table):
    seq = inputs.shape[1]
    s_rows = seq // 4
    partial = _sc_broadcast_add(inputs, position_table, s_rows)
    return _tc_fill_rest(inputs, position_table, partial, s_rows)


# speedup vs baseline: 1.3295x; 1.3295x over previous
import functools

import jax
import jax.numpy as jnp
from jax import lax
from jax.experimental import pallas as pl
from jax.experimental.pallas import tpu as pltpu
from jax.experimental.pallas import tpu_sc as plsc

_P = 16
_BS = 512


def _sc_broadcast_add(inputs, table, s_rows):
    batch, seq, dim = inputs.shape
    info = plsc.get_sparse_core_info()
    nc, ns, nl = info.num_cores, info.num_subcores, info.num_lanes
    nw = nc * ns
    s_per_w = s_rows // nw
    n_chunks = s_per_w // _P
    n_iter = n_chunks * batch
    mesh = plsc.VectorSubcoreMesh(core_axis_name="c", subcore_axis_name="s")

    @functools.partial(
        pl.kernel,
        mesh=mesh,
        out_type=jax.ShapeDtypeStruct((batch, seq, dim), jnp.float32),
        scratch_types=(
            [pltpu.VMEM((_P, dim), jnp.float32) for _ in range(5)]
            + [pltpu.VMEM((_P, dim), jnp.float32) for _ in range(2)]
            + [pltpu.SemaphoreType.DMA for _ in range(12)]
        ),
    )
    def k(in_hbm, tab_hbm, out_hbm, *scr):
        bufs = scr[0:5]
        tabs = scr[5:7]
        lsem = scr[7:12]
        ssem = scr[12:17]
        tsem = scr[17:19]
        wid = lax.axis_index("s") * nc + lax.axis_index("c")
        s0 = wid * s_per_w

        def in_slice(i):
            c, b = divmod(i, batch)
            return in_hbm.at[b, pl.ds(s0 + c * _P, _P)]

        def out_slice(i):
            c, b = divmod(i, batch)
            return out_hbm.at[b, pl.ds(s0 + c * _P, _P)]

        def tab_slice(c):
            return tab_hbm.at[pl.ds(s0 + c * _P, _P)]

        def load(i):
            return pltpu.make_async_copy(in_slice(i), bufs[i % 5], lsem[i % 5])

        def store(i):
            return pltpu.make_async_copy(bufs[i % 5], out_slice(i), ssem[i % 5])

        def tabcp(c):
            return pltpu.make_async_copy(tab_slice(c), tabs[c % 2], tsem[c % 2])

        tabcp(0).start()
        if n_chunks > 1:
            tabcp(1).start()
        for i in range(min(4, n_iter)):
            load(i).start()
        for i in range(n_iter):
            c, b = divmod(i, batch)
            load(i).wait()
            if b == 0:
                tabcp(c).wait()
            buf = bufs[i % 5]
            tv = tabs[c % 2]

            def body(i2, carry, buf=buf, tv=tv):
                r = i2 // 2
                col = (i2 % 2) * (dim // 2)
                for u in range(dim // (2 * nl)):
                    sl = pl.ds(col + u * nl, nl)
                    buf[r, sl] = buf[r, sl] + tv[r, sl]
                return carry

            lax.fori_loop(0, 2 * _P, body, 0)
            store(i).start()
            if b == batch - 1 and c + 2 < n_chunks:
                tabcp(c + 2).start()
            if i + 4 < n_iter:
                if i >= 1:
                    store(i - 1).wait()
                load(i + 4).start()
        for i in range(max(0, n_iter - 5), n_iter):
            store(i).wait()

    return k(inputs, table)


def _tc_fill_rest(inputs, table, partial, row0):
    batch, seq, dim = inputs.shape
    rows = seq - row0
    blk0 = row0 // _BS

    def body(in_ref, tab_ref, alias_ref, out_ref):
        del alias_ref
        out_ref[...] = in_ref[...] + tab_ref[...]

    return pl.pallas_call(
        body,
        grid=(rows // _BS, batch),
        in_specs=[
            pl.BlockSpec((1, _BS, dim), lambda i, b: (b, i + blk0, 0)),
            pl.BlockSpec((_BS, dim), lambda i, b: (i + blk0, 0)),
            pl.BlockSpec(memory_space=pl.ANY),
        ],
        out_specs=pl.BlockSpec((1, _BS, dim), lambda i, b: (b, i + blk0, 0)),
        out_shape=jax.ShapeDtypeStruct((batch, seq, dim), jnp.float32),
        input_output_aliases={2: 0},
    )(inputs, table, partial)


def kernel(inputs, position_table):
    seq = inputs.shape[1]
    s_rows = seq // 4
    partial = _sc_broadcast_add(inputs, position_table, s_rows)
    return _tc_fill_rest(inputs, position_table, partial, s_rows)


# speedup vs baseline: 1.3901x; 1.0456x over previous
import functools

import jax
import jax.numpy as jnp
from jax import lax
from jax.experimental import pallas as pl
from jax.experimental.pallas import tpu as pltpu
from jax.experimental.pallas import tpu_sc as plsc

_P = 16
_BS = 1024


def _sc_broadcast_add(inputs, table, s_rows):
    batch, seq, dim = inputs.shape
    info = plsc.get_sparse_core_info()
    nc, ns, nl = info.num_cores, info.num_subcores, info.num_lanes
    nw = nc * ns
    s_per_w = s_rows // nw
    n_chunks = s_per_w // _P
    n_iter = n_chunks * batch
    mesh = plsc.VectorSubcoreMesh(core_axis_name="c", subcore_axis_name="s")

    @functools.partial(
        pl.kernel,
        mesh=mesh,
        out_type=jax.ShapeDtypeStruct((batch, seq, dim), jnp.float32),
        scratch_types=(
            [pltpu.VMEM((_P, dim), jnp.float32) for _ in range(5)]
            + [pltpu.VMEM((_P, dim), jnp.float32) for _ in range(2)]
            + [pltpu.SemaphoreType.DMA for _ in range(12)]
        ),
    )
    def k(in_hbm, tab_hbm, out_hbm, *scr):
        bufs = scr[0:5]
        tabs = scr[5:7]
        lsem = scr[7:12]
        ssem = scr[12:17]
        tsem = scr[17:19]
        wid = lax.axis_index("s") * nc + lax.axis_index("c")
        s0 = wid * s_per_w

        def in_slice(i):
            c, b = divmod(i, batch)
            return in_hbm.at[b, pl.ds(s0 + c * _P, _P)]

        def out_slice(i):
            c, b = divmod(i, batch)
            return out_hbm.at[b, pl.ds(s0 + c * _P, _P)]

        def tab_slice(c):
            return tab_hbm.at[pl.ds(s0 + c * _P, _P)]

        def load(i):
            return pltpu.make_async_copy(in_slice(i), bufs[i % 5], lsem[i % 5])

        def store(i):
            return pltpu.make_async_copy(bufs[i % 5], out_slice(i), ssem[i % 5])

        def tabcp(c):
            return pltpu.make_async_copy(tab_slice(c), tabs[c % 2], tsem[c % 2])

        tabcp(0).start()
        if n_chunks > 1:
            tabcp(1).start()
        for i in range(min(4, n_iter)):
            load(i).start()
        for i in range(n_iter):
            c, b = divmod(i, batch)
            load(i).wait()
            if b == 0:
                tabcp(c).wait()
            buf = bufs[i % 5]
            tv = tabs[c % 2]

            def body(i2, carry, buf=buf, tv=tv):
                r = i2 // 2
                col = (i2 % 2) * (dim // 2)
                for u in range(dim // (2 * nl)):
                    sl = pl.ds(col + u * nl, nl)
                    buf[r, sl] = buf[r, sl] + tv[r, sl]
                return carry

            lax.fori_loop(0, 2 * _P, body, 0)
            store(i).start()
            if b == batch - 1 and c + 2 < n_chunks:
                tabcp(c + 2).start()
            if i + 4 < n_iter:
                if i >= 1:
                    store(i - 1).wait()
                load(i + 4).start()
        for i in range(max(0, n_iter - 5), n_iter):
            store(i).wait()

    return k(inputs, table)


def _tc_fill_rest(inputs, table, partial, row0):
    batch, seq, dim = inputs.shape
    rows = seq - row0
    blk0 = row0 // _BS

    def body(in_ref, tab_ref, alias_ref, out_ref):
        del alias_ref
        out_ref[...] = in_ref[...] + tab_ref[...]

    return pl.pallas_call(
        body,
        grid=(rows // _BS, batch),
        in_specs=[
            pl.BlockSpec((1, _BS, dim), lambda i, b: (b, i + blk0, 0)),
            pl.BlockSpec((_BS, dim), lambda i, b: (i + blk0, 0)),
            pl.BlockSpec(memory_space=pl.ANY),
        ],
        out_specs=pl.BlockSpec((1, _BS, dim), lambda i, b: (b, i + blk0, 0)),
        out_shape=jax.ShapeDtypeStruct((batch, seq, dim), jnp.float32),
        input_output_aliases={2: 0},
    )(inputs, table, partial)


def kernel(inputs, position_table):
    seq = inputs.shape[1]
    s_rows = seq // 4
    partial = _sc_broadcast_add(inputs, position_table, s_rows)
    return _tc_fill_rest(inputs, position_table, partial, s_rows)
